# baseline (speedup 1.0000x reference)
import functools
import jax
import jax.numpy as jnp
from jax import lax
from jax.experimental import pallas as pl
from jax.experimental.pallas import tpu as pltpu
from jax.experimental.pallas import tpu_sc as plsc

KK = 5
NN = 10000
EE = 160000
NPAD = 10240
T = 128
EPAD = 168960
NT = EPAD // T
NC = 2
NS = 16
NW = NC * NS
WCHUNK = EPAD // NW
CH = 120
NSUB = WCHUNK // CH
D = 128


def _sc_gather(h, src_pad):
    mesh = plsc.VectorSubcoreMesh(core_axis_name="c", subcore_axis_name="s")

    @functools.partial(
        pl.kernel, mesh=mesh,
        out_type=jax.ShapeDtypeStruct((EPAD, D), jnp.float32),
        scratch_types=[
            pltpu.VMEM((CH,), jnp.int32),
            pltpu.VMEM((CH, D), jnp.float32),
            pltpu.SemaphoreType.DMA,
        ],
    )
    def k(h_hbm, src_hbm, out_hbm, idx_v, rows_v, sem):
        wid = lax.axis_index("s") * NC + lax.axis_index("c")
        base = wid * WCHUNK

        def body(j, carry):
            off = base + j * CH
            pltpu.sync_copy(src_hbm.at[pl.ds(off, CH)], idx_v)
            pltpu.async_copy(h_hbm.at[idx_v], rows_v, sem).wait()
            pltpu.sync_copy(rows_v, out_hbm.at[pl.ds(off, CH)])
            return carry

        lax.fori_loop(0, NSUB, body, 0)

    return k(h, src_pad)


def _sc_scatter(m, dst_pad, zero_init):
    mesh = plsc.VectorSubcoreMesh(core_axis_name="c", subcore_axis_name="s")

    @functools.partial(
        pl.kernel, mesh=mesh,
        out_type=jax.ShapeDtypeStruct((NC, NPAD, D), jnp.float32),
        scratch_types=[
            pltpu.VMEM((CH,), jnp.int32),
            pltpu.VMEM((CH, D), jnp.float32),
            pltpu.VMEM_SHARED((NPAD, D), jnp.float32),
        ],
    )
    def k(m_hbm, dst_hbm, zero_hbm, out_hbm, idx_v, m_v, table_sh):
        c = lax.axis_index("c")
        s = lax.axis_index("s")
        wid = s * NC + c

        @pl.when(s == 0)
        def _():
            pltpu.sync_copy(zero_hbm, table_sh)

        plsc.subcore_barrier()
        base = wid * WCHUNK

        def body(j, carry):
            off = base + j * CH
            pltpu.sync_copy(dst_hbm.at[pl.ds(off, CH)], idx_v)
            pltpu.sync_copy(m_hbm.at[pl.ds(off, CH)], m_v)
            pltpu.sync_copy(m_v, table_sh.at[idx_v], add=True)
            return carry

        lax.fori_loop(0, NSUB, body, 0)
        plsc.subcore_barrier()
        rows = NPAD // NS
        pltpu.sync_copy(table_sh.at[pl.ds(s * rows, rows)],
                        out_hbm.at[c, pl.ds(s * rows, rows)])

    return k(m, dst_pad, zero_init)


def _tc_messages(tile_cell, ea_pad, xj, valid, wc2, cin, cout):
    def body(tc_ref, ea_ref, xj_ref, val_ref, wc_ref, out_ref):
        ea = ea_ref[...]
        u = jnp.clip(ea, 0.0, 1.0) * (KK - 1)
        lo = jnp.clip(jnp.floor(u), 0.0, KK - 2.0)
        f = u - lo
        xjv = xj_ref[...][:, :cin]
        cols = []
        for b in range(8):
            w = jnp.ones((T, 1), jnp.float32)
            for d in range(3):
                fd = f[:, d:d + 1]
                w = w * (fd if (b >> d) & 1 else (1.0 - fd))
            cols.append(xjv * w)
        x8 = jnp.concatenate(cols, axis=1)
        acc = jnp.dot(x8, wc_ref[0], preferred_element_type=jnp.float32)
        acc = acc * val_ref[...]
        if cout < D:
            acc = jnp.concatenate(
                [acc, jnp.zeros((T, D - cout), jnp.float32)], axis=1)
        out_ref[...] = acc

    grid_spec = pltpu.PrefetchScalarGridSpec(
        num_scalar_prefetch=1,
        grid=(NT,),
        in_specs=[
            pl.BlockSpec((T, 3), lambda i, tc: (i, 0)),
            pl.BlockSpec((T, D), lambda i, tc: (i, 0)),
            pl.BlockSpec((T, 1), lambda i, tc: (i, 0)),
            pl.BlockSpec((1, 8 * cin, cout), lambda i, tc: (tc[i], 0, 0)),
        ],
        out_specs=pl.BlockSpec((T, D), lambda i, tc: (i, 0)),
    )
    return pl.pallas_call(
        body, grid_spec=grid_spec,
        out_shape=jax.ShapeDtypeStruct((EPAD, D), jnp.float32),
    )(tile_cell, ea_pad, xj, valid, wc2)


def _tc_normalize(agg2, deg2, hprev, rootp, biasr, cout):
    TN = 256

    def body(a_ref, d_ref, h_ref, r_ref, b_ref, o_ref):
        a = a_ref[0] + a_ref[1]
        deg = (d_ref[0] + d_ref[1])[:, 0:1]
        z = (a[:, :cout] / jnp.maximum(deg, 1.0)
             + jnp.dot(h_ref[...], r_ref[...],
                       preferred_element_type=jnp.float32)
             + b_ref[...])
        z = jnp.where(z > 0, z, jnp.exp(z) - 1.0)
        if cout < D:
            z = jnp.concatenate(
                [z, jnp.zeros((TN, D - cout), jnp.float32)], axis=1)
        o_ref[...] = z

    return pl.pallas_call(
        body,
        grid=(NPAD // TN,),
        in_specs=[
            pl.BlockSpec((2, TN, D), lambda i: (0, i, 0)),
            pl.BlockSpec((2, TN, D), lambda i: (0, i, 0)),
            pl.BlockSpec((TN, D), lambda i: (i, 0)),
            pl.BlockSpec((D, cout), lambda i: (0, 0)),
            pl.BlockSpec((1, cout), lambda i: (0, 0)),
        ],
        out_specs=pl.BlockSpec((TN, D), lambda i: (i, 0)),
        out_shape=jax.ShapeDtypeStruct((NPAD, D), jnp.float32),
    )(agg2, deg2, hprev, rootp, biasr)


def _tc_head(h, vox_pad, fc1w3, fc1b, fc2wp, fc2bp):
    def body(h_ref, v_ref, w1_ref, b1_ref, w2_ref, b2_ref, o_ref):
        hv = h_ref[...]
        vox = v_ref[...]
        iot = lax.broadcasted_iota(jnp.int32, (NPAD, 8), 1)
        oh = vox == iot
        z = b1_ref[...]
        for v in range(8):
            col = oh[:, v:v + 1]
            pv = jnp.max(jnp.where(col, hv, -jnp.inf), axis=0, keepdims=True)
            pv = jnp.where(jnp.isfinite(pv), pv, 0.0)
            z = z + jnp.dot(pv, w1_ref[v], preferred_element_type=jnp.float32)
        z = jnp.where(z > 0, z, jnp.exp(z) - 1.0)
        z2 = jnp.dot(z, w2_ref[...], preferred_element_type=jnp.float32) + b2_ref[...]
        mask = lax.broadcasted_iota(jnp.int32, (1, 128), 1) < 10
        mx = jnp.max(jnp.where(mask, z2, -jnp.inf))
        lse = jnp.log(jnp.sum(jnp.where(mask, jnp.exp(z2 - mx), 0.0))) + mx
        o_ref[...] = z2 - lse

    return pl.pallas_call(
        body,
        out_shape=jax.ShapeDtypeStruct((1, 128), jnp.float32),
    )(h, vox_pad, fc1w3, fc1b, fc2wp, fc2bp)


def _corner_table(W, ci_pad):
    ci = W.shape[1]
    co = W.shape[2]
    Wp = jnp.pad(W, ((0, 0), (0, ci_pad - ci), (0, 0)))
    l0, l1, l2 = jnp.meshgrid(jnp.arange(4), jnp.arange(4), jnp.arange(4),
                              indexing='ij')
    cells = jnp.stack([l0.ravel(), l1.ravel(), l2.ravel()], 1)
    idx = []
    for b in range(8):
        bits = [(b >> d) & 1 for d in range(3)]
        idx.append(((cells[:, 0] + bits[0]) * KK + (cells[:, 1] + bits[1])) * KK
                   + (cells[:, 2] + bits[2]))
    idx = jnp.stack(idx, 1)
    return Wp[idx].reshape(64, 8 * ci_pad, co)


def kernel(x, edge_index, edge_attr, voxel8, W1, root1, b1, W2, root2, b2,
           W3, root3, b3, W4, root4, b4, fc1_w, fc1_b, fc2_w, fc2_b):
    src, dst = edge_index[0], edge_index[1]
    u = jnp.clip(edge_attr, 0.0, 1.0) * (KK - 1)
    lo = jnp.clip(jnp.floor(u), 0, KK - 2).astype(jnp.int32)
    cell = (lo[:, 0] * 4 + lo[:, 1]) * 4 + lo[:, 2]
    order = jnp.argsort(cell)
    counts = jnp.bincount(cell, length=64)
    starts = jnp.cumsum(counts) - counts
    tcnt = -(-counts // T)
    toff = (jnp.cumsum(tcnt) - tcnt) * T
    cs = cell[order]
    rank = jnp.arange(EE) - starts[cs]
    pos = toff[cs] + rank
    src_pad = jnp.zeros(EPAD, jnp.int32).at[pos].set(src[order])
    dst_pad = jnp.zeros(EPAD, jnp.int32).at[pos].set(dst[order])
    ea_pad = jnp.zeros((EPAD, 3), jnp.float32).at[pos].set(edge_attr[order])
    valid = jnp.zeros((EPAD, 1), jnp.float32).at[pos].set(1.0)
    tile_cell = jnp.minimum(
        jnp.repeat(jnp.arange(64, dtype=jnp.int32), tcnt,
                   total_repeat_length=NT), 63)

    zero_tbl = jnp.zeros((NPAD, D), jnp.float32)
    deg2 = _sc_scatter(jnp.broadcast_to(valid, (EPAD, D)), dst_pad, zero_tbl)

    h = jnp.pad(x, ((0, NPAD - NN), (0, D - 1)))
    layers = [(8, 32, W1, root1, b1), (32, 64, W2, root2, b2),
              (64, 64, W3, root3, b3), (64, 128, W4, root4, b4)]
    for ci, co, Wl, rootl, bl in layers:
        wc2 = _corner_table(Wl, ci)
        rootp = jnp.pad(rootl, ((0, D - rootl.shape[0]), (0, 0)))
        xj = _sc_gather(h, src_pad)
        m = _tc_messages(tile_cell, ea_pad, xj, valid, wc2, ci, co)
        agg2 = _sc_scatter(m, dst_pad, zero_tbl)
        h = _tc_normalize(agg2, deg2, h, rootp, bl[None, :], co)

    vox_pad = jnp.pad(voxel8, (0, NPAD - NN), constant_values=8)[:, None]
    out = _tc_head(h, vox_pad, fc1_w.reshape(8, 128, 256), fc1_b[None, :],
                   jnp.pad(fc2_w, ((0, 0), (0, 118))),
                   jnp.pad(fc2_b, (0, 118))[None, :])
    return out[:, :10]
